# baseline (speedup 1.0000x reference)
import jax
import jax.numpy as jnp
from jax import lax
from jax.experimental import pallas as pl
from jax.experimental.pallas import tpu as pltpu
from jax.experimental.pallas import tpu_sc as plsc

HID = 64
N = 2048
KCLIP = 64
NW = 32
TB = 4096
CPW = TB // NW


def _band_body(emb, a_out, tbl, stage, csem, wsem):
    c = lax.axis_index("c")
    s = lax.axis_index("s")
    wid = s * 2 + c
    t0 = wid * CPW
    cp = pltpu.make_async_copy(emb, tbl, csem)
    cp.start()
    cp.wait()

    def row(w, carry):
        idx = jnp.clip(t0 + w - (N - 1 - KCLIP), 0, 2 * KCLIP)
        for cc in range(HID // 16):
            stage[pl.ds(w * HID + cc * 16, 16)] = tbl[idx, pl.ds(cc * 16, 16)]
        return carry

    lax.fori_loop(0, CPW, row, 0)
    cp = pltpu.make_async_copy(stage, a_out.at[pl.ds(t0 * HID, CPW * HID)], wsem)
    cp.start()
    cp.wait()


def _mat_body(a_ref, o_hbm, rolled, sem):
    c = pl.program_id(0)
    par = c % 2

    def waits(p):
        for _ in range(16):
            pltpu.make_async_copy(
                rolled.at[0, :, pl.ds(0, N)], o_hbm.at[0], sem.at[p]
            ).wait()

    @pl.when(c >= 2)
    def _():
        waits(par)

    x = a_ref[...]
    y = pltpu.roll(x, TB - c, axis=1)
    rolled[par] = y
    for k in range(16):
        i_k = 127 - c + 128 * k
        s2 = 128 * (15 - k)
        pltpu.make_async_copy(
            rolled.at[par, :, pl.ds(s2, N)], o_hbm.at[i_k], sem.at[par]
        ).start()

    @pl.when(c == 127)
    def _():
        waits(0)
        waits(1)


def kernel(embeddings, q_len, k_len):
    band = pl.kernel(
        _band_body,
        out_type=jax.ShapeDtypeStruct((TB * HID,), jnp.float32),
        mesh=plsc.VectorSubcoreMesh(core_axis_name="c", subcore_axis_name="s"),
        compiler_params=pltpu.CompilerParams(use_tc_tiling_on_sc=False),
        scratch_types=[
            pltpu.VMEM((2 * KCLIP + 1, HID), jnp.float32),
            pltpu.VMEM((CPW * HID,), jnp.float32),
            pltpu.SemaphoreType.DMA,
            pltpu.SemaphoreType.DMA,
        ],
    )
    a = band(embeddings)
    a_t = jnp.transpose(jnp.reshape(a, (TB, HID)))
    out_t = pl.pallas_call(
        _mat_body,
        grid=(128,),
        in_specs=[pl.BlockSpec((HID, TB), lambda i: (0, 0))],
        out_specs=pl.BlockSpec(memory_space=pltpu.HBM),
        out_shape=jax.ShapeDtypeStruct((N, HID, N), jnp.float32),
        scratch_shapes=[
            pltpu.VMEM((2, HID, TB), jnp.float32),
            pltpu.SemaphoreType.DMA((2,)),
        ],
    )(a_t)
    return jnp.transpose(out_t, (0, 2, 1))
